# Initial kernel scaffold; baseline (speedup 1.0000x reference)
import jax
import jax.numpy as jnp
from jax.experimental import pallas as pl


def kernel(x, edge_index, edge_weight, h0, W_lin, b_lin, w_ih, w_hh, b_ih, b_hh, convW, convb, W_fc, b_fc):
    raise NotImplementedError("write your pallas kernel here")


# speedup vs baseline: 6.0989x; 6.0989x over previous
import functools

import jax
import jax.numpy as jnp
from jax import lax
from jax.experimental import pallas as pl
from jax.experimental.pallas import tpu as pltpu
from jax.experimental.pallas import tpu_sc as plsc

N = 10000
F = 128
HG = 128
C = 40
E = 320000

NC = 2
NS = 16
CH = 128
KCH = 80
EPT = CH * KCH
EPAD = NC * NS * EPT
NPAD = 10240
RPT = NPAD // NS

_mesh = plsc.VectorSubcoreMesh(core_axis_name="c", subcore_axis_name="s")


def _sc_deg_body(dst_hbm, ew_hbm, z1_hbm, out_hbm, dst_v, ew_v, acc, sem):
    c = lax.axis_index("c")
    s = lax.axis_index("s")
    pltpu.sync_copy(z1_hbm, acc.at[pl.ds(s * RPT, RPT)])
    pltpu.sync_copy(dst_hbm.at[c, s], dst_v)
    pltpu.sync_copy(ew_hbm.at[c, s], ew_v)
    plsc.subcore_barrier()

    def chunk(k, carry):
        pltpu.sync_copy(ew_v.at[k], acc.at[dst_v.at[k]], add=True)
        return carry

    lax.fori_loop(0, KCH, chunk, 0)
    plsc.subcore_barrier()
    pltpu.sync_copy(acc.at[pl.ds(s * RPT, RPT)], out_hbm.at[c, pl.ds(s * RPT, RPT)])


def _sc_deg(dst4, ew4, z1):
    kfn = pl.kernel(
        _sc_deg_body,
        out_type=jax.ShapeDtypeStruct((NC, NPAD), jnp.float32),
        mesh=_mesh,
        scratch_types=[
            pltpu.VMEM((KCH, CH), jnp.int32),
            pltpu.VMEM((KCH, CH), jnp.float32),
            pltpu.VMEM_SHARED((NPAD,), jnp.float32),
            pltpu.SemaphoreType.DMA,
        ],
    )
    return kfn(dst4, ew4, z1)


def _sc_conv_body(ys_hbm, src_hbm, dst_hbm, ew_hbm, z2_hbm, out_hbm,
                  src_v, dst_v, ew_v, rows_v, acc, sem):
    c = lax.axis_index("c")
    s = lax.axis_index("s")
    pltpu.sync_copy(z2_hbm, acc.at[pl.ds(s * RPT, RPT // 2)])
    pltpu.sync_copy(z2_hbm, acc.at[pl.ds(s * RPT + RPT // 2, RPT // 2)])
    pltpu.sync_copy(src_hbm.at[c, s], src_v)
    pltpu.sync_copy(dst_hbm.at[c, s], dst_v)
    pltpu.sync_copy(ew_hbm.at[c, s], ew_v)
    plsc.subcore_barrier()

    def chunk(k, carry):
        pltpu.async_copy(ys_hbm.at[src_v.at[k]], rows_v, sem).wait()

        def scale(e, carry2):
            i16 = jnp.full((16,), k * CH + e, dtype=jnp.int32)
            w16 = plsc.load_gather(ew_v, [i16])
            for f in range(F // 16):
                sl = pl.ds(f * 16, 16)
                rows_v[e, sl] = rows_v[e, sl] * w16
            return carry2

        lax.fori_loop(0, CH, scale, 0)
        pltpu.sync_copy(rows_v, acc.at[dst_v.at[k]], add=True)
        return carry

    lax.fori_loop(0, KCH, chunk, 0)
    plsc.subcore_barrier()
    pltpu.sync_copy(acc.at[pl.ds(s * RPT, RPT)], out_hbm.at[c, pl.ds(s * RPT, RPT)])


def _sc_conv(ys, src4, dst4, ew4, z2):
    kfn = pl.kernel(
        _sc_conv_body,
        out_type=jax.ShapeDtypeStruct((NC, NPAD, F), jnp.float32),
        mesh=_mesh,
        scratch_types=[
            pltpu.VMEM((KCH, CH), jnp.int32),
            pltpu.VMEM((KCH, CH), jnp.int32),
            pltpu.VMEM((EPT,), jnp.float32),
            pltpu.VMEM((CH, F), jnp.float32),
            pltpu.VMEM_SHARED((NPAD, F), jnp.float32),
            pltpu.SemaphoreType.DMA,
        ],
        compiler_params=pltpu.CompilerParams(needs_layout_passes=False),
    )
    return kfn(ys, src4, dst4, ew4, z2)


B = 1000

_dn = (((1,), (1,)), ((), ()))


def _gru(x, h, wih, whh, bih, bhh):
    gi = lax.dot_general(x, wih, _dn, preferred_element_type=jnp.float32) + bih
    gh = lax.dot_general(h, whh, _dn, preferred_element_type=jnp.float32) + bhh
    i_r, i_z, i_n = gi[:, :HG], gi[:, HG:2 * HG], gi[:, 2 * HG:]
    h_r, h_z, h_n = gh[:, :HG], gh[:, HG:2 * HG], gh[:, 2 * HG:]
    r = jax.nn.sigmoid(i_r + h_r)
    z = jax.nn.sigmoid(i_z + h_z)
    n = jnp.tanh(i_n + r * h_n)
    return (1.0 - z) * n + z * h


def _stage0_body(x_ref, h0_ref, dinv_ref, wlin_ref, blin_ref, wih_ref, whh_ref,
                 bih_ref, bhh_ref, cw_ref, h_out, ys_out):
    x = x_ref[...]
    x1 = jnp.maximum(
        lax.dot_general(x, wlin_ref[...], _dn, preferred_element_type=jnp.float32)
        + blin_ref[...], 0.0)
    h_out[...] = _gru(x1, h0_ref[...], wih_ref[...], whh_ref[...],
                      bih_ref[...], bhh_ref[...])
    ys_out[...] = dinv_ref[...] * lax.dot_general(
        x1, cw_ref[...], _dn, preferred_element_type=jnp.float32)


def _stage_mid_body(agg_ref, ys_ref, dinv_ref, b_ref, h_ref, wih_ref, whh_ref,
                    bih_ref, bhh_ref, cw_ref, h_out, ys_out):
    aggsum = agg_ref[0] + agg_ref[1] + ys_ref[...]
    x = jnp.maximum(dinv_ref[...] * aggsum + b_ref[...], 0.0)
    h_out[...] = _gru(x, h_ref[...], wih_ref[...], whh_ref[...],
                      bih_ref[...], bhh_ref[...])
    ys_out[...] = dinv_ref[...] * lax.dot_general(
        x, cw_ref[...], _dn, preferred_element_type=jnp.float32)


def _stage_last_body(agg_ref, ys_ref, dinv_ref, b_ref, h_ref, wih_ref, whh_ref,
                     bih_ref, bhh_ref, wfc_ref, bfc_ref, o_out):
    aggsum = agg_ref[0] + agg_ref[1] + ys_ref[...]
    x = jnp.maximum(dinv_ref[...] * aggsum + b_ref[...], 0.0)
    h = _gru(x, h_ref[...], wih_ref[...], whh_ref[...], bih_ref[...], bhh_ref[...])
    o_out[...] = (lax.dot_general(h, wfc_ref[...], _dn,
                                  preferred_element_type=jnp.float32)
                  + bfc_ref[...])


def _row_spec(w):
    return pl.BlockSpec((B, w), lambda i: (i, 0))


def _full_spec(shape):
    nd = len(shape)
    return pl.BlockSpec(shape, lambda i: (0,) * nd)


def _stage0(x, h0, dinv, wlin, blin, wih, whh, bih, bhh, cw0):
    grid = (N // B,)
    return pl.pallas_call(
        _stage0_body,
        grid=grid,
        in_specs=[
            _row_spec(F), _row_spec(HG), _row_spec(1),
            _full_spec((F, F)), _full_spec((1, F)),
            _full_spec((3 * HG, F)), _full_spec((3 * HG, HG)),
            _full_spec((1, 3 * HG)), _full_spec((1, 3 * HG)),
            _full_spec((F, F)),
        ],
        out_specs=[_row_spec(HG), _row_spec(F)],
        out_shape=[jax.ShapeDtypeStruct((N, HG), jnp.float32),
                   jax.ShapeDtypeStruct((N, F), jnp.float32)],
    )(x, h0, dinv, wlin, blin, wih, whh, bih, bhh, cw0)


def _agg_spec():
    return pl.BlockSpec((NC, B, F), lambda i: (0, i, 0))


def _stage_mid(agg, ys, dinv, b, h, wih, whh, bih, bhh, cw):
    grid = (N // B,)
    return pl.pallas_call(
        _stage_mid_body,
        grid=grid,
        in_specs=[
            _agg_spec(), _row_spec(F), _row_spec(1), _full_spec((1, F)),
            _row_spec(HG),
            _full_spec((3 * HG, F)), _full_spec((3 * HG, HG)),
            _full_spec((1, 3 * HG)), _full_spec((1, 3 * HG)),
            _full_spec((F, F)),
        ],
        out_specs=[_row_spec(HG), _row_spec(F)],
        out_shape=[jax.ShapeDtypeStruct((N, HG), jnp.float32),
                   jax.ShapeDtypeStruct((N, F), jnp.float32)],
    )(agg, ys, dinv, b, h, wih, whh, bih, bhh, cw)


def _stage_last(agg, ys, dinv, b, h, wih, whh, bih, bhh, wfc, bfc):
    grid = (N // B,)
    return pl.pallas_call(
        _stage_last_body,
        grid=grid,
        in_specs=[
            _agg_spec(), _row_spec(F), _row_spec(1), _full_spec((1, F)),
            _row_spec(HG),
            _full_spec((3 * HG, F)), _full_spec((3 * HG, HG)),
            _full_spec((1, 3 * HG)), _full_spec((1, 3 * HG)),
            _full_spec((C, HG)), _full_spec((1, C)),
        ],
        out_specs=[_row_spec(C)],
        out_shape=[jax.ShapeDtypeStruct((N, C), jnp.float32)],
    )(agg, ys, dinv, b, h, wih, whh, bih, bhh, wfc, bfc)[0]


def kernel(x, edge_index, edge_weight, h0, W_lin, b_lin, w_ih, w_hh, b_ih, b_hh,
           convW, convb, W_fc, b_fc):
    src = edge_index[0]
    dst = edge_index[1]
    pad = EPAD - E
    src4 = jnp.concatenate([src, jnp.zeros((pad,), jnp.int32)]).reshape(NC, NS, KCH, CH)
    dst4 = jnp.concatenate([dst, jnp.zeros((pad,), jnp.int32)]).reshape(NC, NS, KCH, CH)
    ew_pad = jnp.concatenate([edge_weight, jnp.zeros((pad,), jnp.float32)])
    ew4 = ew_pad.reshape(NC, NS, KCH, CH)
    ew3 = ew_pad.reshape(NC, NS, EPT)
    z1 = jnp.zeros((RPT,), jnp.float32)
    z2 = jnp.zeros((RPT // 2, F), jnp.float32)

    degp = _sc_deg(dst4, ew4, z1)
    deg = degp[0, :N] + degp[1, :N] + 1.0
    dinv = lax.rsqrt(deg).reshape(N, 1)

    blin = b_lin.reshape(1, F)
    bih = b_ih.reshape(1, 3 * HG)
    bhh = b_hh.reshape(1, 3 * HG)
    bfc = b_fc.reshape(1, C)

    h, ys = _stage0(x, h0, dinv, W_lin, blin, w_ih, w_hh, bih, bhh, convW[0])
    for i in range(3):
        agg = _sc_conv(ys, src4, dst4, ew3, z2)
        bi = convb[i].reshape(1, F)
        if i < 2:
            h, ys = _stage_mid(agg, ys, dinv, bi, h, w_ih, w_hh, bih, bhh,
                               convW[i + 1])
        else:
            out = _stage_last(agg, ys, dinv, bi, h, w_ih, w_hh, bih, bhh,
                              W_fc, bfc)
    return out


# speedup vs baseline: 6.1008x; 1.0003x over previous
import functools

import jax
import jax.numpy as jnp
from jax import lax
from jax.experimental import pallas as pl
from jax.experimental.pallas import tpu as pltpu
from jax.experimental.pallas import tpu_sc as plsc

N = 10000
F = 128
HG = 128
C = 40
E = 320000

NC = 2
NS = 16
CH = 128
KCH = 80
EPT = CH * KCH
EPAD = NC * NS * EPT
NPAD = 10240
RPT = NPAD // NS

_mesh = plsc.VectorSubcoreMesh(core_axis_name="c", subcore_axis_name="s")


def _sc_deg_body(dst_hbm, ew_hbm, z1_hbm, out_hbm, dst_v, ew_v, acc, sem):
    c = lax.axis_index("c")
    s = lax.axis_index("s")
    pltpu.sync_copy(z1_hbm, acc.at[pl.ds(s * RPT, RPT)])
    pltpu.sync_copy(dst_hbm.at[c, s], dst_v)
    pltpu.sync_copy(ew_hbm.at[c, s], ew_v)
    plsc.subcore_barrier()

    def chunk(k, carry):
        pltpu.sync_copy(ew_v.at[k], acc.at[dst_v.at[k]], add=True)
        return carry

    lax.fori_loop(0, KCH, chunk, 0)
    plsc.subcore_barrier()
    pltpu.sync_copy(acc.at[pl.ds(s * RPT, RPT)], out_hbm.at[c, pl.ds(s * RPT, RPT)])


def _sc_deg(dst4, ew4, z1):
    kfn = pl.kernel(
        _sc_deg_body,
        out_type=jax.ShapeDtypeStruct((NC, NPAD), jnp.float32),
        mesh=_mesh,
        scratch_types=[
            pltpu.VMEM((KCH, CH), jnp.int32),
            pltpu.VMEM((KCH, CH), jnp.float32),
            pltpu.VMEM_SHARED((NPAD,), jnp.float32),
            pltpu.SemaphoreType.DMA,
        ],
    )
    return kfn(dst4, ew4, z1)


def _sc_conv_body(ys_hbm, src_hbm, dst_hbm, ew_hbm, z2_hbm, out_hbm,
                  src_v, dst_v, ew_v, r0, r1, r2, r3, acc, g0, g1, g2, g3):
    rowsl = (r0, r1, r2, r3)
    gseml = (g0, g1, g2, g3)
    c = lax.axis_index("c")
    s = lax.axis_index("s")
    pltpu.sync_copy(z2_hbm, acc.at[pl.ds(s * RPT, RPT // 2)])
    pltpu.sync_copy(z2_hbm, acc.at[pl.ds(s * RPT + RPT // 2, RPT // 2)])
    pltpu.sync_copy(src_hbm.at[c, s], src_v)
    pltpu.sync_copy(dst_hbm.at[c, s], dst_v)
    pltpu.sync_copy(ew_hbm.at[c, s], ew_v)
    plsc.subcore_barrier()

    def chunk(k, carry):
        rows_v = rowsl[0]
        pltpu.async_copy(ys_hbm.at[src_v.at[k]], rows_v, gseml[0]).wait()

        def scale(e, carry2):
            i16 = jnp.full((16,), k * CH + e, dtype=jnp.int32)
            w16 = plsc.load_gather(ew_v, [i16])
            for f in range(F // 16):
                sl = pl.ds(f * 16, 16)
                rows_v[e, sl] = rows_v[e, sl] * w16
            return carry2

        lax.fori_loop(0, CH, scale, 0)
        pltpu.sync_copy(rows_v, acc.at[dst_v.at[k]], add=True)
        return carry

    lax.fori_loop(0, KCH, chunk, 0)
    plsc.subcore_barrier()
    pltpu.sync_copy(acc.at[pl.ds(s * RPT, RPT)], out_hbm.at[c, pl.ds(s * RPT, RPT)])


def _sc_conv(ys, src4, dst4, ew4, z2):
    kfn = pl.kernel(
        _sc_conv_body,
        out_type=jax.ShapeDtypeStruct((NC, NPAD, F), jnp.float32),
        mesh=_mesh,
        scratch_types=[
            pltpu.VMEM((KCH, CH), jnp.int32),
            pltpu.VMEM((KCH, CH), jnp.int32),
            pltpu.VMEM((EPT,), jnp.float32),
            pltpu.VMEM((CH, F), jnp.float32),
            pltpu.VMEM((CH, F), jnp.float32),
            pltpu.VMEM((CH, F), jnp.float32),
            pltpu.VMEM((CH, F), jnp.float32),
            pltpu.VMEM_SHARED((NPAD, F), jnp.float32),
            pltpu.SemaphoreType.DMA,
            pltpu.SemaphoreType.DMA,
            pltpu.SemaphoreType.DMA,
            pltpu.SemaphoreType.DMA,
        ],
        compiler_params=pltpu.CompilerParams(needs_layout_passes=False),
    )
    return kfn(ys, src4, dst4, ew4, z2)


B = 1000

_dn = (((1,), (1,)), ((), ()))


def _gru(x, h, wih, whh, bih, bhh):
    gi = lax.dot_general(x, wih, _dn, preferred_element_type=jnp.float32) + bih
    gh = lax.dot_general(h, whh, _dn, preferred_element_type=jnp.float32) + bhh
    i_r, i_z, i_n = gi[:, :HG], gi[:, HG:2 * HG], gi[:, 2 * HG:]
    h_r, h_z, h_n = gh[:, :HG], gh[:, HG:2 * HG], gh[:, 2 * HG:]
    r = jax.nn.sigmoid(i_r + h_r)
    z = jax.nn.sigmoid(i_z + h_z)
    n = jnp.tanh(i_n + r * h_n)
    return (1.0 - z) * n + z * h


def _stage0_body(x_ref, h0_ref, dinv_ref, wlin_ref, blin_ref, wih_ref, whh_ref,
                 bih_ref, bhh_ref, cw_ref, h_out, ys_out):
    x = x_ref[...]
    x1 = jnp.maximum(
        lax.dot_general(x, wlin_ref[...], _dn, preferred_element_type=jnp.float32)
        + blin_ref[...], 0.0)
    h_out[...] = _gru(x1, h0_ref[...], wih_ref[...], whh_ref[...],
                      bih_ref[...], bhh_ref[...])
    ys_out[...] = dinv_ref[...] * lax.dot_general(
        x1, cw_ref[...], _dn, preferred_element_type=jnp.float32)


def _stage_mid_body(agg_ref, ys_ref, dinv_ref, b_ref, h_ref, wih_ref, whh_ref,
                    bih_ref, bhh_ref, cw_ref, h_out, ys_out):
    aggsum = agg_ref[0] + agg_ref[1] + ys_ref[...]
    x = jnp.maximum(dinv_ref[...] * aggsum + b_ref[...], 0.0)
    h_out[...] = _gru(x, h_ref[...], wih_ref[...], whh_ref[...],
                      bih_ref[...], bhh_ref[...])
    ys_out[...] = dinv_ref[...] * lax.dot_general(
        x, cw_ref[...], _dn, preferred_element_type=jnp.float32)


def _stage_last_body(agg_ref, ys_ref, dinv_ref, b_ref, h_ref, wih_ref, whh_ref,
                     bih_ref, bhh_ref, wfc_ref, bfc_ref, o_out):
    aggsum = agg_ref[0] + agg_ref[1] + ys_ref[...]
    x = jnp.maximum(dinv_ref[...] * aggsum + b_ref[...], 0.0)
    h = _gru(x, h_ref[...], wih_ref[...], whh_ref[...], bih_ref[...], bhh_ref[...])
    o_out[...] = (lax.dot_general(h, wfc_ref[...], _dn,
                                  preferred_element_type=jnp.float32)
                  + bfc_ref[...])


def _row_spec(w):
    return pl.BlockSpec((B, w), lambda i: (i, 0))


def _full_spec(shape):
    nd = len(shape)
    return pl.BlockSpec(shape, lambda i: (0,) * nd)


def _stage0(x, h0, dinv, wlin, blin, wih, whh, bih, bhh, cw0):
    grid = (N // B,)
    return pl.pallas_call(
        _stage0_body,
        grid=grid,
        in_specs=[
            _row_spec(F), _row_spec(HG), _row_spec(1),
            _full_spec((F, F)), _full_spec((1, F)),
            _full_spec((3 * HG, F)), _full_spec((3 * HG, HG)),
            _full_spec((1, 3 * HG)), _full_spec((1, 3 * HG)),
            _full_spec((F, F)),
        ],
        out_specs=[_row_spec(HG), _row_spec(F)],
        out_shape=[jax.ShapeDtypeStruct((N, HG), jnp.float32),
                   jax.ShapeDtypeStruct((N, F), jnp.float32)],
    )(x, h0, dinv, wlin, blin, wih, whh, bih, bhh, cw0)


def _agg_spec():
    return pl.BlockSpec((NC, B, F), lambda i: (0, i, 0))


def _stage_mid(agg, ys, dinv, b, h, wih, whh, bih, bhh, cw):
    grid = (N // B,)
    return pl.pallas_call(
        _stage_mid_body,
        grid=grid,
        in_specs=[
            _agg_spec(), _row_spec(F), _row_spec(1), _full_spec((1, F)),
            _row_spec(HG),
            _full_spec((3 * HG, F)), _full_spec((3 * HG, HG)),
            _full_spec((1, 3 * HG)), _full_spec((1, 3 * HG)),
            _full_spec((F, F)),
        ],
        out_specs=[_row_spec(HG), _row_spec(F)],
        out_shape=[jax.ShapeDtypeStruct((N, HG), jnp.float32),
                   jax.ShapeDtypeStruct((N, F), jnp.float32)],
    )(agg, ys, dinv, b, h, wih, whh, bih, bhh, cw)


def _stage_last(agg, ys, dinv, b, h, wih, whh, bih, bhh, wfc, bfc):
    grid = (N // B,)
    return pl.pallas_call(
        _stage_last_body,
        grid=grid,
        in_specs=[
            _agg_spec(), _row_spec(F), _row_spec(1), _full_spec((1, F)),
            _row_spec(HG),
            _full_spec((3 * HG, F)), _full_spec((3 * HG, HG)),
            _full_spec((1, 3 * HG)), _full_spec((1, 3 * HG)),
            _full_spec((C, HG)), _full_spec((1, C)),
        ],
        out_specs=[_row_spec(C)],
        out_shape=[jax.ShapeDtypeStruct((N, C), jnp.float32)],
    )(agg, ys, dinv, b, h, wih, whh, bih, bhh, wfc, bfc)[0]


def kernel(x, edge_index, edge_weight, h0, W_lin, b_lin, w_ih, w_hh, b_ih, b_hh,
           convW, convb, W_fc, b_fc):
    src = edge_index[0]
    dst = edge_index[1]
    pad = EPAD - E
    src4 = jnp.concatenate([src, jnp.zeros((pad,), jnp.int32)]).reshape(NC, NS, KCH, CH)
    dst4 = jnp.concatenate([dst, jnp.zeros((pad,), jnp.int32)]).reshape(NC, NS, KCH, CH)
    ew_pad = jnp.concatenate([edge_weight, jnp.zeros((pad,), jnp.float32)])
    ew4 = ew_pad.reshape(NC, NS, KCH, CH)
    ew3 = ew_pad.reshape(NC, NS, EPT)
    z1 = jnp.zeros((RPT,), jnp.float32)
    z2 = jnp.zeros((RPT // 2, F), jnp.float32)

    degp = _sc_deg(dst4, ew4, z1)
    deg = degp[0, :N] + degp[1, :N] + 1.0
    dinv = lax.rsqrt(deg).reshape(N, 1)

    blin = b_lin.reshape(1, F)
    bih = b_ih.reshape(1, 3 * HG)
    bhh = b_hh.reshape(1, 3 * HG)
    bfc = b_fc.reshape(1, C)

    h, ys = _stage0(x, h0, dinv, W_lin, blin, w_ih, w_hh, bih, bhh, convW[0])
    for i in range(3):
        agg = _sc_conv(ys, src4, dst4, ew3, z2)
        bi = convb[i].reshape(1, F)
        if i < 2:
            h, ys = _stage_mid(agg, ys, dinv, bi, h, w_ih, w_hh, bih, bhh,
                               convW[i + 1])
        else:
            out = _stage_last(agg, ys, dinv, bi, h, w_ih, w_hh, bih, bhh,
                              W_fc, bfc)
    return out
